# initial kernel scaffold (unmeasured)
import jax
import jax.numpy as jnp
from jax import lax
from jax.experimental import pallas as pl
from jax.experimental.pallas import tpu as pltpu

N_DEV = 4
SQ = 128
HQ = 8
HKV = 2
DH = 128
D = 1024
GROUP = HQ // HKV
GROWS = GROUP * SQ
SCALE = 0.08838834764831843
CHUNK = 1024
ML = DH + 128


def kernel(x, Wq, Wo, K_ext, V_ext):
    skv_local = K_ext.shape[1]
    n_chunks = skv_local // CHUNK

    def body(x_ref, wq_ref, wo_ref, k_ref, v_ref, out_ref,
             q_ref, o_ref, m_ref, l_ref, comm_ref, send_sems, recv_sems):
        g = pl.program_id(0)
        j = pl.program_id(1)

        @pl.when((g == 0) & (j == 0))
        def _init():
            xs = x_ref[...]
            for h in range(HQ):
                qh = jnp.dot(xs, wq_ref[:, h * DH:(h + 1) * DH],
                             preferred_element_type=jnp.float32)
                r0 = (h % GROUP) * SQ
                q_ref[h // GROUP, r0:r0 + SQ, :] = qh * SCALE
            o_ref[...] = jnp.zeros(o_ref.shape, jnp.float32)
            l_ref[...] = jnp.zeros(l_ref.shape, jnp.float32)
            m_ref[...] = jnp.full(m_ref.shape, -jnp.inf, jnp.float32)

        def update(gi):
            q = q_ref[gi]
            s = lax.dot_general(q, k_ref[...], (((1,), (1,)), ((), ())),
                                preferred_element_type=jnp.float32)
            mj = jnp.max(s, axis=-1, keepdims=True)
            m_old = m_ref[gi]
            m_new = jnp.maximum(m_old, mj)
            alpha = jnp.exp(m_old - m_new)
            p = jnp.exp(s - m_new)
            l_ref[gi] = l_ref[gi] * alpha + jnp.sum(p, axis=-1, keepdims=True)
            o_ref[gi] = o_ref[gi] * alpha + lax.dot_general(
                p, v_ref[...], (((1,), (0,)), ((), ())),
                preferred_element_type=jnp.float32)
            m_ref[gi] = m_new

        @pl.when(g == 0)
        def _():
            update(0)

        @pl.when(g == 1)
        def _():
            update(1)

        @pl.when((g == HKV - 1) & (j == n_chunks - 1))
        def _ring():
            my = lax.axis_index("i")
            right = (my + 1) % N_DEV
            left = (my + N_DEV - 1) % N_DEV

            comm_ref[0, :, :, 0:DH] = o_ref[...]
            comm_ref[0, :, :, DH:DH + 1] = m_ref[...]
            comm_ref[0, :, :, DH + 1:DH + 2] = l_ref[...]

            barrier = pltpu.get_barrier_semaphore()
            for nbr in (left, right):
                pl.semaphore_signal(barrier, inc=1, device_id=(nbr,),
                                    device_id_type=pl.DeviceIdType.MESH)
            pl.semaphore_wait(barrier, 2)

            for h in range(N_DEV - 1):
                rdma = pltpu.make_async_remote_copy(
                    src_ref=comm_ref.at[h],
                    dst_ref=comm_ref.at[h + 1],
                    send_sem=send_sems.at[h],
                    recv_sem=recv_sems.at[h + 1],
                    device_id=(right,),
                    device_id_type=pl.DeviceIdType.MESH,
                )
                rdma.start()
                rdma.wait()

                o_r = comm_ref[h + 1, :, :, 0:DH]
                m_r = comm_ref[h + 1, :, :, DH:DH + 1]
                l_r = comm_ref[h + 1, :, :, DH + 1:DH + 2]
                m_old = m_ref[...]
                m_new = jnp.maximum(m_old, m_r)
                a_old = jnp.exp(m_old - m_new)
                a_r = jnp.exp(m_r - m_new)
                o_ref[...] = o_ref[...] * a_old + o_r * a_r
                l_ref[...] = l_ref[...] * a_old + l_r * a_r
                m_ref[...] = m_new

            attn = o_ref[...] / l_ref[...]
            acc = jnp.zeros((SQ, D), jnp.float32)
            for h in range(HQ):
                r0 = (h % GROUP) * SQ
                ah = attn[h // GROUP, r0:r0 + SQ, :]
                acc = acc + jnp.dot(ah, wo_ref[h * DH:(h + 1) * DH, :],
                                    preferred_element_type=jnp.float32)
            out_ref[...] = acc

    return pl.pallas_call(
        body,
        grid=(HKV, n_chunks),
        in_specs=[
            pl.BlockSpec((None, SQ, D), lambda g, j: (0, 0, 0)),
            pl.BlockSpec((D, D), lambda g, j: (0, 0)),
            pl.BlockSpec((D, D), lambda g, j: (0, 0)),
            pl.BlockSpec((None, CHUNK, None, DH), lambda g, j: (0, j, g, 0)),
            pl.BlockSpec((None, CHUNK, None, DH), lambda g, j: (0, j, g, 0)),
        ],
        out_specs=pl.BlockSpec((None, SQ, D), lambda g, j: (0, 0, 0)),
        out_shape=jax.ShapeDtypeStruct((1, SQ, D), jnp.float32),
        scratch_shapes=[
            pltpu.VMEM((HKV, GROWS, DH), jnp.float32),
            pltpu.VMEM((HKV, GROWS, DH), jnp.float32),
            pltpu.VMEM((HKV, GROWS, 1), jnp.float32),
            pltpu.VMEM((HKV, GROWS, 1), jnp.float32),
            pltpu.VMEM((N_DEV, HKV, GROWS, ML), jnp.float32),
            pltpu.SemaphoreType.DMA((N_DEV,)),
            pltpu.SemaphoreType.DMA((N_DEV,)),
        ],
        compiler_params=pltpu.CompilerParams(
            dimension_semantics=("arbitrary", "arbitrary"),
            collective_id=0,
        ),
    )(x, Wq, Wo, K_ext, V_ext)


# baseline (device time: 167086 ns/iter reference)
import jax
import jax.numpy as jnp
from jax import lax
from jax.experimental import pallas as pl
from jax.experimental.pallas import tpu as pltpu

N_DEV = 4
SQ = 128
HQ = 8
HKV = 2
DH = 128
D = 1024
GROUP = HQ // HKV
GROWS = GROUP * SQ
SCALE = 0.08838834764831843
CHUNK = 1024
ML = DH + 128


def kernel(x, Wq, Wo, K_ext, V_ext):
    skv_local = K_ext.shape[1]
    n_chunks = skv_local // CHUNK
    K2 = K_ext.reshape(skv_local, HKV * DH)
    V2 = V_ext.reshape(skv_local, HKV * DH)

    def body(x_ref, wq_ref, wo_ref, k_ref, v_ref, out_ref,
             q_ref, o_ref, m_ref, l_ref, comm_ref, send_sems, recv_sems):
        j = pl.program_id(0)

        @pl.when(j == 0)
        def _init():
            xs = x_ref[...]
            for h in range(HQ):
                qh = jnp.dot(xs, wq_ref[:, h * DH:(h + 1) * DH],
                             preferred_element_type=jnp.float32)
                r0 = (h % GROUP) * SQ
                q_ref[h // GROUP, r0:r0 + SQ, :] = qh * SCALE
            o_ref[...] = jnp.zeros(o_ref.shape, jnp.float32)
            l_ref[...] = jnp.zeros(l_ref.shape, jnp.float32)
            m_ref[...] = jnp.full(m_ref.shape, -jnp.inf, jnp.float32)

        for gi in range(HKV):
            q = q_ref[gi]
            kg = k_ref[:, gi * DH:(gi + 1) * DH]
            vg = v_ref[:, gi * DH:(gi + 1) * DH]
            s = lax.dot_general(q, kg, (((1,), (1,)), ((), ())),
                                preferred_element_type=jnp.float32)
            mj = jnp.max(s, axis=-1, keepdims=True)
            m_old = m_ref[gi]
            m_new = jnp.maximum(m_old, mj)
            alpha = jnp.exp(m_old - m_new)
            p = jnp.exp(s - m_new)
            l_ref[gi] = l_ref[gi] * alpha + jnp.sum(p, axis=-1, keepdims=True)
            o_ref[gi] = o_ref[gi] * alpha + lax.dot_general(
                p, vg, (((1,), (0,)), ((), ())),
                preferred_element_type=jnp.float32)
            m_ref[gi] = m_new

        @pl.when(j == n_chunks - 1)
        def _ring():
            my = lax.axis_index("i")
            right = (my + 1) % N_DEV
            left = (my + N_DEV - 1) % N_DEV

            comm_ref[0, :, :, 0:DH] = o_ref[...]
            comm_ref[0, :, :, DH:DH + 1] = m_ref[...]
            comm_ref[0, :, :, DH + 1:DH + 2] = l_ref[...]

            barrier = pltpu.get_barrier_semaphore()
            for nbr in (left, right):
                pl.semaphore_signal(barrier, inc=1, device_id=(nbr,),
                                    device_id_type=pl.DeviceIdType.MESH)
            pl.semaphore_wait(barrier, 2)

            for h in range(N_DEV - 1):
                rdma = pltpu.make_async_remote_copy(
                    src_ref=comm_ref.at[h],
                    dst_ref=comm_ref.at[h + 1],
                    send_sem=send_sems.at[h],
                    recv_sem=recv_sems.at[h + 1],
                    device_id=(right,),
                    device_id_type=pl.DeviceIdType.MESH,
                )
                rdma.start()
                rdma.wait()

                o_r = comm_ref[h + 1, :, :, 0:DH]
                m_r = comm_ref[h + 1, :, :, DH:DH + 1]
                l_r = comm_ref[h + 1, :, :, DH + 1:DH + 2]
                m_old = m_ref[...]
                m_new = jnp.maximum(m_old, m_r)
                a_old = jnp.exp(m_old - m_new)
                a_r = jnp.exp(m_r - m_new)
                o_ref[...] = o_ref[...] * a_old + o_r * a_r
                l_ref[...] = l_ref[...] * a_old + l_r * a_r
                m_ref[...] = m_new

            attn = o_ref[...] / l_ref[...]
            acc = jnp.zeros((SQ, D), jnp.float32)
            for h in range(HQ):
                r0 = (h % GROUP) * SQ
                ah = attn[h // GROUP, r0:r0 + SQ, :]
                acc = acc + jnp.dot(ah, wo_ref[h * DH:(h + 1) * DH, :],
                                    preferred_element_type=jnp.float32)
            out_ref[...] = acc

    return pl.pallas_call(
        body,
        grid=(n_chunks,),
        in_specs=[
            pl.BlockSpec((None, SQ, D), lambda j: (0, 0, 0)),
            pl.BlockSpec((D, D), lambda j: (0, 0)),
            pl.BlockSpec((D, D), lambda j: (0, 0)),
            pl.BlockSpec((CHUNK, HKV * DH), lambda j: (j, 0)),
            pl.BlockSpec((CHUNK, HKV * DH), lambda j: (j, 0)),
        ],
        out_specs=pl.BlockSpec((None, SQ, D), lambda j: (0, 0, 0)),
        out_shape=jax.ShapeDtypeStruct((1, SQ, D), jnp.float32),
        scratch_shapes=[
            pltpu.VMEM((HKV, GROWS, DH), jnp.float32),
            pltpu.VMEM((HKV, GROWS, DH), jnp.float32),
            pltpu.VMEM((HKV, GROWS, 1), jnp.float32),
            pltpu.VMEM((HKV, GROWS, 1), jnp.float32),
            pltpu.VMEM((N_DEV, HKV, GROWS, ML), jnp.float32),
            pltpu.SemaphoreType.DMA((N_DEV,)),
            pltpu.SemaphoreType.DMA((N_DEV,)),
        ],
        compiler_params=pltpu.CompilerParams(
            dimension_semantics=("arbitrary",),
            collective_id=0,
        ),
    )(x, Wq, Wo, K2, V2)


# device time: 126901 ns/iter; 1.3167x vs baseline; 1.3167x over previous
import jax
import jax.numpy as jnp
from jax import lax
from jax.experimental import pallas as pl
from jax.experimental.pallas import tpu as pltpu

N_DEV = 4
SQ = 128
HQ = 8
HKV = 2
DH = 128
D = 1024
GROUP = HQ // HKV
GROWS = GROUP * SQ
SCALE = 0.08838834764831843
CHUNK = 1024
ML = DH + 128
NSLOT = 4


def kernel(x, Wq, Wo, K_ext, V_ext):
    skv_local = K_ext.shape[1]
    n_chunks = skv_local // CHUNK

    def body(x_ref, wq_ref, wo_ref, k_hbm, v_hbm, out_ref,
             q_ref, o_ref, m_ref, l_ref, kbuf, vbuf, kv_sems,
             comm_ref, send_sems, recv_sems):
        j = pl.program_id(0)

        def kv_copies(chunk, slot):
            ops = []
            for tensor_i, (hbm, buf) in enumerate(((k_hbm, kbuf), (v_hbm, vbuf))):
                for gi in range(HKV):
                    ops.append(pltpu.make_async_copy(
                        hbm.at[0, pl.ds(chunk * CHUNK, CHUNK), gi, :],
                        buf.at[slot, gi],
                        kv_sems.at[slot, tensor_i * HKV + gi],
                    ))
            return ops

        @pl.when(j == 0)
        def _init():
            for c in (0, 1):
                for op in kv_copies(c, c):
                    op.start()
            xs = x_ref[...]
            for h in range(HQ):
                qh = jnp.dot(xs, wq_ref[:, h * DH:(h + 1) * DH],
                             preferred_element_type=jnp.float32)
                r0 = (h % GROUP) * SQ
                q_ref[h // GROUP, r0:r0 + SQ, :] = qh * SCALE
            o_ref[...] = jnp.zeros(o_ref.shape, jnp.float32)
            l_ref[...] = jnp.zeros(l_ref.shape, jnp.float32)
            m_ref[...] = jnp.full(m_ref.shape, -jnp.inf, jnp.float32)

        @pl.when(j + 2 < n_chunks)
        def _prefetch():
            for op in kv_copies(j + 2, (j + 2) % NSLOT):
                op.start()

        slot = j % NSLOT
        for op in kv_copies(j, slot):
            op.wait()

        for gi in range(HKV):
            q = q_ref[gi]
            kg = kbuf[slot, gi]
            vg = vbuf[slot, gi]
            s = lax.dot_general(q, kg, (((1,), (1,)), ((), ())),
                                preferred_element_type=jnp.float32)
            mj = jnp.max(s, axis=-1, keepdims=True)
            m_old = m_ref[gi]
            m_new = jnp.maximum(m_old, mj)
            alpha = jnp.exp(m_old - m_new)
            p = jnp.exp(s - m_new)
            l_ref[gi] = l_ref[gi] * alpha + jnp.sum(p, axis=-1, keepdims=True)
            o_ref[gi] = o_ref[gi] * alpha + lax.dot_general(
                p, vg, (((1,), (0,)), ((), ())),
                preferred_element_type=jnp.float32)
            m_ref[gi] = m_new

        @pl.when(j == n_chunks - 1)
        def _ring():
            my = lax.axis_index("i")
            right = (my + 1) % N_DEV
            left = (my + N_DEV - 1) % N_DEV

            comm_ref[0, :, :, 0:DH] = o_ref[...]
            comm_ref[0, :, :, DH:DH + 1] = m_ref[...]
            comm_ref[0, :, :, DH + 1:DH + 2] = l_ref[...]

            barrier = pltpu.get_barrier_semaphore()
            for nbr in (left, right):
                pl.semaphore_signal(barrier, inc=1, device_id=(nbr,),
                                    device_id_type=pl.DeviceIdType.MESH)
            pl.semaphore_wait(barrier, 2)

            for h in range(N_DEV - 1):
                rdma = pltpu.make_async_remote_copy(
                    src_ref=comm_ref.at[h],
                    dst_ref=comm_ref.at[h + 1],
                    send_sem=send_sems.at[h],
                    recv_sem=recv_sems.at[h + 1],
                    device_id=(right,),
                    device_id_type=pl.DeviceIdType.MESH,
                )
                rdma.start()
                rdma.wait()

                o_r = comm_ref[h + 1, :, :, 0:DH]
                m_r = comm_ref[h + 1, :, :, DH:DH + 1]
                l_r = comm_ref[h + 1, :, :, DH + 1:DH + 2]
                m_old = m_ref[...]
                m_new = jnp.maximum(m_old, m_r)
                a_old = jnp.exp(m_old - m_new)
                a_r = jnp.exp(m_r - m_new)
                o_ref[...] = o_ref[...] * a_old + o_r * a_r
                l_ref[...] = l_ref[...] * a_old + l_r * a_r
                m_ref[...] = m_new

            attn = o_ref[...] / l_ref[...]
            acc = jnp.zeros((SQ, D), jnp.float32)
            for h in range(HQ):
                r0 = (h % GROUP) * SQ
                ah = attn[h // GROUP, r0:r0 + SQ, :]
                acc = acc + jnp.dot(ah, wo_ref[h * DH:(h + 1) * DH, :],
                                    preferred_element_type=jnp.float32)
            out_ref[...] = acc

    return pl.pallas_call(
        body,
        grid=(n_chunks,),
        in_specs=[
            pl.BlockSpec((None, SQ, D), lambda j: (0, 0, 0)),
            pl.BlockSpec((D, D), lambda j: (0, 0)),
            pl.BlockSpec((D, D), lambda j: (0, 0)),
            pl.BlockSpec(memory_space=pltpu.MemorySpace.HBM),
            pl.BlockSpec(memory_space=pltpu.MemorySpace.HBM),
        ],
        out_specs=pl.BlockSpec((None, SQ, D), lambda j: (0, 0, 0)),
        out_shape=jax.ShapeDtypeStruct((1, SQ, D), jnp.float32),
        scratch_shapes=[
            pltpu.VMEM((HKV, GROWS, DH), jnp.float32),
            pltpu.VMEM((HKV, GROWS, DH), jnp.float32),
            pltpu.VMEM((HKV, GROWS, 1), jnp.float32),
            pltpu.VMEM((HKV, GROWS, 1), jnp.float32),
            pltpu.VMEM((NSLOT, HKV, CHUNK, DH), jnp.float32),
            pltpu.VMEM((NSLOT, HKV, CHUNK, DH), jnp.float32),
            pltpu.SemaphoreType.DMA((NSLOT, 2 * HKV)),
            pltpu.VMEM((N_DEV, HKV, GROWS, ML), jnp.float32),
            pltpu.SemaphoreType.DMA((N_DEV,)),
            pltpu.SemaphoreType.DMA((N_DEV,)),
        ],
        compiler_params=pltpu.CompilerParams(
            dimension_semantics=("arbitrary",),
            collective_id=0,
        ),
    )(x, Wq, Wo, K_ext, V_ext)
